# baseline (speedup 1.0000x reference)
import jax
import jax.numpy as jnp
from jax.experimental import pallas as pl
from jax.experimental.pallas import tpu as pltpu

_SUB = 2048
_K = 4


def _matmul_bias_kernel(*refs):
    x_refs = refs[:_K]
    w_ref, b_ref, o_ref = refs[_K], refs[_K + 1], refs[_K + 2]
    w = w_ref[...]
    b = b_ref[...]
    for j in range(_K):
        o_ref[pl.ds(j * _SUB, _SUB), :] = (
            jnp.dot(x_refs[j][...], w, preferred_element_type=jnp.float32) + b
        )


def kernel(input, W, b):
    n, in_f = input.shape
    out_f = W.shape[1]
    b2 = b.reshape(1, out_f)
    step_rows = _SUB * _K
    grid = (n // step_rows,)

    def make_spec(j):
        return pl.BlockSpec((_SUB, in_f), lambda i, j=j: (_K * i + j, 0))

    out = pl.pallas_call(
        _matmul_bias_kernel,
        grid=grid,
        in_specs=[make_spec(j) for j in range(_K)] + [
            pl.BlockSpec((in_f, out_f), lambda i: (0, 0)),
            pl.BlockSpec((1, out_f), lambda i: (0, 0)),
        ],
        out_specs=pl.BlockSpec((step_rows, out_f), lambda i: (i, 0)),
        out_shape=jax.ShapeDtypeStruct((n, out_f), jnp.float32),
        compiler_params=pltpu.CompilerParams(
            dimension_semantics=("arbitrary",),
        ),
    )(*([input] * _K + [W, b2]))
    return out
